# baseline (speedup 1.0000x reference)
import jax
import jax.numpy as jnp
from jax.experimental import pallas as pl
from jax.experimental.pallas import tpu as pltpu

_B, _T, _D = 4, 8192, 1024
_H = 16
_HD = 64
_K = 128
_FF = 2816


def _decoder_body(idx_ref, hid_ref, cos_ref, sin_ref,
                  Wq, bq, Wk, bk, Wv, bv, Wo, ln1, ln2, Wg, Wu, Wd,
                  out_ref, sel_scr, cos_scr, sin_scr, sem_h, sem_c, sem_s):
    b = pl.program_id(0)

    def issue(k, carry):
        row = idx_ref[b, k]
        pltpu.make_async_copy(hid_ref.at[b, pl.ds(row, 1), :],
                              sel_scr.at[pl.ds(k, 1), :], sem_h).start()
        pltpu.make_async_copy(cos_ref.at[b, pl.ds(row, 1), :],
                              cos_scr.at[pl.ds(k, 1), :], sem_c).start()
        pltpu.make_async_copy(sin_ref.at[b, pl.ds(row, 1), :],
                              sin_scr.at[pl.ds(k, 1), :], sem_s).start()
        return carry

    jax.lax.fori_loop(0, _K, issue, 0)

    def drain(k, carry):
        pltpu.make_async_copy(hid_ref.at[b, pl.ds(0, 1), :],
                              sel_scr.at[pl.ds(k, 1), :], sem_h).wait()
        pltpu.make_async_copy(cos_ref.at[b, pl.ds(0, 1), :],
                              cos_scr.at[pl.ds(k, 1), :], sem_c).wait()
        pltpu.make_async_copy(sin_ref.at[b, pl.ds(0, 1), :],
                              sin_scr.at[pl.ds(k, 1), :], sem_s).wait()
        return carry

    jax.lax.fori_loop(0, _K, drain, 0)

    sel = sel_scr[...]
    cosv = cos_scr[...]
    sinv = sin_scr[...]

    def rms(x, w):
        v = jnp.mean(x * x, axis=-1, keepdims=True)
        return x * jax.lax.rsqrt(v + 1e-6) * w

    def mm(x, w):
        return jax.lax.dot_general(
            x.astype(jnp.bfloat16), w, (((1,), (0,)), ((), ())),
            preferred_element_type=jnp.float32)

    h = rms(sel, ln1[...])
    q = mm(h, Wq[...]) + bq[...]
    kk = mm(h, Wk[...]) + bk[...]
    v = mm(h, Wv[...]) + bv[...]

    def rope(x):
        x1 = x[:, :_HD // 2]
        x2 = x[:, _HD // 2:]
        rh = jnp.concatenate([-x2, x1], axis=1)
        return x * cosv + rh * sinv

    row_i = jax.lax.broadcasted_iota(jnp.int32, (_K, _K), 0)
    col_i = jax.lax.broadcasted_iota(jnp.int32, (_K, _K), 1)
    causal = col_i <= row_i
    neg = jnp.finfo(jnp.float32).min

    o_parts = []
    for hh in range(_H):
        sl = slice(hh * _HD, (hh + 1) * _HD)
        qh = rope(q[:, sl])
        kh = rope(kk[:, sl])
        vh = v[:, sl]
        s = jax.lax.dot_general(
            qh.astype(jnp.bfloat16), kh.astype(jnp.bfloat16),
            (((1,), (1,)), ((), ())), preferred_element_type=jnp.float32)
        s = s * (1.0 / (_HD ** 0.5))
        s = jnp.where(causal, s, neg)
        m = jnp.max(s, axis=-1, keepdims=True)
        p = jnp.exp(s - m)
        p = p / jnp.sum(p, axis=-1, keepdims=True)
        oh = jax.lax.dot_general(
            p.astype(jnp.bfloat16), vh.astype(jnp.bfloat16),
            (((1,), (0,)), ((), ())), preferred_element_type=jnp.float32)
        o_parts.append(oh)
    o = jnp.concatenate(o_parts, axis=1)

    h1 = sel + mm(o, Wo[...])
    h2 = rms(h1, ln2[...])
    g = mm(h2, Wg[...])
    u = mm(h2, Wu[...])
    act = g * (1.0 / (1.0 + jnp.exp(-g))) * u
    out = h1 + mm(act, Wd[...])
    out_ref[0] = out


def kernel(hidden_states, topk_indices, cos, sin, Wq, bq, Wk, bk, Wv, bv, Wo,
           ln1_w, ln2_w, Wgate, Wup, Wdown):
    B, T, D = hidden_states.shape
    K = topk_indices.shape[1]
    idx = topk_indices.astype(jnp.int32)

    wbf = lambda w: w.astype(jnp.bfloat16)
    row = lambda x: x.reshape(1, -1)

    vm_full = lambda shape: pl.BlockSpec(shape, lambda b, s: (0,) * len(shape))
    any_spec = pl.BlockSpec(memory_space=pl.ANY)

    processed = pl.pallas_call(
        _decoder_body,
        grid_spec=pltpu.PrefetchScalarGridSpec(
            num_scalar_prefetch=1,
            grid=(B,),
            in_specs=[
                any_spec, any_spec, any_spec,
                vm_full((D, D)), vm_full((1, D)),
                vm_full((D, D)), vm_full((1, D)),
                vm_full((D, D)), vm_full((1, D)),
                vm_full((D, D)),
                vm_full((1, D)), vm_full((1, D)),
                vm_full((D, _FF)), vm_full((D, _FF)), vm_full((_FF, D)),
            ],
            out_specs=pl.BlockSpec((1, K, D), lambda b, s: (b, 0, 0)),
            scratch_shapes=[
                pltpu.VMEM((K, D), jnp.float32),
                pltpu.VMEM((K, _HD), jnp.float32),
                pltpu.VMEM((K, _HD), jnp.float32),
                pltpu.SemaphoreType.DMA,
                pltpu.SemaphoreType.DMA,
                pltpu.SemaphoreType.DMA,
            ],
        ),
        out_shape=jax.ShapeDtypeStruct((B, K, D), jnp.float32),
    )(idx, hidden_states, cos, sin,
      wbf(Wq), row(bq), wbf(Wk), row(bk), wbf(Wv), row(bv), wbf(Wo),
      row(ln1_w), row(ln2_w), wbf(Wgate), wbf(Wup), wbf(Wdown))
    return processed
